# baseline (speedup 1.0000x reference)
import functools

import jax
import jax.numpy as jnp
import numpy as np
from jax import lax
from jax.experimental import pallas as pl
from jax.experimental.pallas import tpu as pltpu
from jax.experimental.pallas import tpu_sc as plsc

N_NODES = 10000
B = 128
AVG_LOG = float(
    (np.log(np.arange(8, dtype=np.float64) + 1.0)
     * np.array([0.0, 5000.0, 10000.0, 15000.0, 10000.0, 5000.0, 3000.0, 2000.0])).sum()
    / 50000.0)

_NC, _NS = 2, 16
_NW = _NC * _NS
_NPAD = 10240
_NPT = _NPAD // _NW
_E = 320000
_CHUNK = 1600
_NCHUNK = _E // _CHUNK
_G = 64
_ACC_ROWS = _NPT + 4
_NEG = -3.0e38
_POS = 3.0e38


def _make_sc_agg(fp):
    acc_n = _ACC_ROWS * fp
    mesh = plsc.VectorSubcoreMesh(core_axis_name="c", subcore_axis_name="s")
    out_type = ([jax.ShapeDtypeStruct((_NPAD * fp,), jnp.float32)] * 4
                + [jax.ShapeDtypeStruct((_NPAD,), jnp.float32)])
    scratch = [
        pltpu.VMEM((_CHUNK,), jnp.int32),
        pltpu.VMEM((_CHUNK,), jnp.int32),
        pltpu.VMEM((_CHUNK + _G,), jnp.int32),
        pltpu.VMEM((_CHUNK + _G,), jnp.int32),
        pltpu.VMEM((_G, 128), jnp.float32),
        pltpu.VMEM((acc_n,), jnp.float32),
        pltpu.VMEM((acc_n,), jnp.float32),
        pltpu.VMEM((acc_n,), jnp.float32),
        pltpu.VMEM((acc_n,), jnp.float32),
        pltpu.VMEM((_NPT + 16,), jnp.float32),
        pltpu.SemaphoreType.DMA,
    ]

    @functools.partial(pl.kernel, out_type=out_type, mesh=mesh,
                       scratch_types=scratch,
                       compiler_params=pltpu.CompilerParams(
                           needs_layout_passes=False))
    def body(b_hbm, src_hbm, dst_hbm, s1_hbm, s2_hbm, mx_hbm, mn_hbm, cnt_hbm,
             src_v, dst_v, msrc, mldst, gbuf, a1, a2, amx, amn, acnt, sem):
        wid = lax.axis_index("s") * _NC + lax.axis_index("c")
        base = wid * _NPT
        zf = jnp.zeros((16,), jnp.float32)
        neg = jnp.full((16,), _NEG, jnp.float32)
        pos = jnp.full((16,), _POS, jnp.float32)
        one0 = jnp.where(lax.iota(jnp.int32, 16) == 0, 1.0, 0.0)

        def init(i, _):
            a1[pl.ds(i * 16, 16)] = zf
            a2[pl.ds(i * 16, 16)] = zf
            amx[pl.ds(i * 16, 16)] = neg
            amn[pl.ds(i * 16, 16)] = pos
            return 0
        lax.fori_loop(0, acc_n // 16, init, 0)

        def initc(i, _):
            acnt[pl.ds(i * 16, 16)] = zf
            return 0
        lax.fori_loop(0, (_NPT + 16) // 16, initc, 0)

        def chunk_body(ci, _):
            pltpu.sync_copy(src_hbm.at[pl.ds(ci * _CHUNK, _CHUNK)], src_v)
            pltpu.sync_copy(dst_hbm.at[pl.ds(ci * _CHUNK, _CHUNK)], dst_v)

            def scan(g, m):
                d = dst_v[pl.ds(g * 16, 16)]
                s = src_v[pl.ds(g * 16, 16)]
                msk = (d >= base) & (d < base + _NPT)
                mi = jnp.where(msk, 1, 0)
                pos = (m + plsc.cumsum(mi)) - mi
                plsc.store_scatter(msrc, [pos], s, mask=msk)
                plsc.store_scatter(mldst, [pos], d - base, mask=msk)
                return m + plsc.all_reduce_population_count(msk)[0]
            m = lax.fori_loop(0, _CHUNK // 16, scan, 0)

            zi = jnp.zeros((16,), jnp.int32)

            def pad(i, _):
                msrc[pl.ds(m + i * 16, 16)] = zi
                return 0
            lax.fori_loop(0, _G // 16, pad, 0)

            def batch(bi, _):
                pltpu.async_copy(
                    b_hbm.at[msrc.at[pl.ds(bi * _G, _G)]], gbuf, sem).wait()
                nr = jnp.minimum(m - bi * _G, _G)

                def row(r, _):
                    ld = mldst[pl.ds(bi * _G + r, 16)][0]
                    off = ld * fp
                    cv = acnt[pl.ds(ld, 16)]
                    acnt[pl.ds(ld, 16)] = cv + one0
                    for c in range(fp // 16):
                        g = gbuf[r, pl.ds(c * 16, 16)]
                        o = off + c * 16
                        a1[pl.ds(o, 16)] = a1[pl.ds(o, 16)] + g
                        a2[pl.ds(o, 16)] = a2[pl.ds(o, 16)] + g * g
                        amx[pl.ds(o, 16)] = jnp.maximum(amx[pl.ds(o, 16)], g)
                        amn[pl.ds(o, 16)] = jnp.minimum(amn[pl.ds(o, 16)], g)
                    return 0
                lax.fori_loop(0, nr, row, 0)
                return 0
            lax.fori_loop(0, (m + _G - 1) // _G, batch, 0)
            return 0
        lax.fori_loop(0, _NCHUNK, chunk_body, 0)

        pltpu.sync_copy(a1.at[pl.ds(0, _NPT * fp)],
                        s1_hbm.at[pl.ds(base * fp, _NPT * fp)])
        pltpu.sync_copy(a2.at[pl.ds(0, _NPT * fp)],
                        s2_hbm.at[pl.ds(base * fp, _NPT * fp)])
        pltpu.sync_copy(amx.at[pl.ds(0, _NPT * fp)],
                        mx_hbm.at[pl.ds(base * fp, _NPT * fp)])
        pltpu.sync_copy(amn.at[pl.ds(0, _NPT * fp)],
                        mn_hbm.at[pl.ds(base * fp, _NPT * fp)])
        pltpu.sync_copy(acnt.at[pl.ds(0, _NPT)], cnt_hbm.at[pl.ds(base, _NPT)])

    return body


_sc_agg = {80: _make_sc_agg(80), 64: _make_sc_agg(64)}


def _pna_layer(x, src, dst, cnt, deg, p):
    f = x.shape[1]
    fp = 80 if f == 78 else 64
    a = x @ p['pre_W'][:f] + p['pre_b']
    b = x @ p['pre_W'][f:]
    bpad = jnp.zeros((_NPAD, 128), jnp.float32).at[:N_NODES, :f].set(b)
    s1f, s2f, mxf, mnf, _ = _sc_agg[fp](bpad, src, dst)
    s1 = s1f.reshape(_NPAD, fp)[:N_NODES, :f]
    s2 = s2f.reshape(_NPAD, fp)[:N_NODES, :f]
    mx = mxf.reshape(_NPAD, fp)[:N_NODES, :f]
    mn = mnf.reshape(_NPAD, fp)[:N_NODES, :f]
    c = cnt[:, None]
    d = deg[:, None]
    mean = (c * a + s1) / d
    mean_sq = (c * a * a + 2.0 * a * s1 + s2) / d
    std = jnp.sqrt(jnp.maximum(mean_sq - mean * mean, 0.0) + 1e-5)
    mxo = jnp.where(c > 0, a + mx, 0.0)
    mno = jnp.where(c > 0, a + mn, 0.0)
    agg = jnp.concatenate([mean, mxo, mno, std], axis=-1)
    logd = jnp.log(deg + 1.0)[:, None]
    w = p['post_W']
    out = (x @ w[:f] + agg @ w[f:f + 4 * f]
           + (logd / AVG_LOG) * (agg @ w[f + 4 * f:f + 8 * f])
           + (AVG_LOG / logd) * (agg @ w[f + 8 * f:])
           + p['post_b'])
    return out @ p['lin_W'] + p['lin_b']


def _bn_relu(h, p):
    return jax.nn.relu(h / np.sqrt(1.0 + 1e-5) * p['bn_g'] + p['bn_b'])


def _conv1d(h, w, b):
    o = jax.lax.conv_general_dilated(h, w, (1,), 'VALID',
                                     dimension_numbers=('NCH', 'OIH', 'NCH'))
    return o + b[None, :, None]


def _mlp_body(xc_ref, w1, b1, w2, b2, w3, b3, w4, b4, out_ref):
    h = jax.nn.relu(jnp.dot(xc_ref[...], w1[...],
                            preferred_element_type=jnp.float32) + b1[...])
    h = jax.nn.relu(jnp.dot(h, w2[...],
                            preferred_element_type=jnp.float32) + b2[...])
    h = jax.nn.relu(jnp.dot(h, w3[...],
                            preferred_element_type=jnp.float32) + b3[...])
    out_ref[...] = jnp.dot(h, w4[...],
                           preferred_element_type=jnp.float32) + b4[...]


def _mlp_head(xc, params):
    return pl.pallas_call(
        _mlp_body,
        out_shape=jax.ShapeDtypeStruct((B, 1), jnp.float32),
    )(xc, params['fc1_W'], params['fc1_b'][None, :],
      params['fc2_W'], params['fc2_b'][None, :],
      params['fc3_W'], params['fc3_b'][None, :],
      params['out_W'], params['out_b'][None, :])


def kernel(x, edge_index, batch, target, params):
    src, dst = edge_index[0], edge_index[1]
    f1 = x.shape[1]
    p1 = params['conv1']
    a1_ = x @ p1['pre_W'][:f1] + p1['pre_b']
    b1_ = x @ p1['pre_W'][f1:]
    bpad = jnp.zeros((_NPAD, 128), jnp.float32).at[:N_NODES, :f1].set(b1_)
    s1f, s2f, mxf, mnf, cntf = _sc_agg[80](bpad, src, dst)
    cnt = cntf[:N_NODES]
    deg = jnp.maximum(cnt, 1.0)
    s1 = s1f.reshape(_NPAD, 80)[:N_NODES, :f1]
    s2 = s2f.reshape(_NPAD, 80)[:N_NODES, :f1]
    mx = mxf.reshape(_NPAD, 80)[:N_NODES, :f1]
    mn = mnf.reshape(_NPAD, 80)[:N_NODES, :f1]
    c = cnt[:, None]
    d = deg[:, None]
    mean = (c * a1_ + s1) / d
    mean_sq = (c * a1_ * a1_ + 2.0 * a1_ * s1 + s2) / d
    std = jnp.sqrt(jnp.maximum(mean_sq - mean * mean, 0.0) + 1e-5)
    mxo = jnp.where(c > 0, a1_ + mx, 0.0)
    mno = jnp.where(c > 0, a1_ + mn, 0.0)
    agg = jnp.concatenate([mean, mxo, mno, std], axis=-1)
    logd = jnp.log(deg + 1.0)[:, None]
    w = p1['post_W']
    h = (x @ w[:f1] + agg @ w[f1:f1 + 4 * f1]
         + (logd / AVG_LOG) * (agg @ w[f1 + 4 * f1:f1 + 8 * f1])
         + (AVG_LOG / logd) * (agg @ w[f1 + 8 * f1:])
         + p1['post_b'])
    h = h @ p1['lin_W'] + p1['lin_b']
    h = _bn_relu(h, p1)
    h = _bn_relu(_pna_layer(h, src, dst, cnt, deg, params['conv2']), params['conv2'])
    h = _bn_relu(_pna_layer(h, src, dst, cnt, deg, params['conv3']), params['conv3'])
    gcnt = jnp.maximum(jax.ops.segment_sum(jnp.ones((batch.shape[0],), jnp.float32),
                                           batch, num_segments=B), 1.0)
    xg = jax.ops.segment_sum(h, batch, num_segments=B) / gcnt[:, None]
    xg = jax.nn.relu(xg @ params['fc1_xd_W'] + params['fc1_xd_b'])
    e = jnp.transpose(params['emb'][target], (0, 2, 1))
    e = jax.nn.relu(_conv1d(e, params['c1_W'], params['c1_b']))
    e = jax.nn.relu(_conv1d(e, params['c2_W'], params['c2_b']))
    e = jax.nn.relu(_conv1d(e, params['c3_W'], params['c3_b']))
    xt = jnp.max(e, axis=2) @ params['pfc_W'] + params['pfc_b']
    xc = jnp.concatenate([xg, xt], axis=1)
    return _mlp_head(xc, params)


# speedup vs baseline: 1.0685x; 1.0685x over previous
import functools

import jax
import jax.numpy as jnp
import numpy as np
from jax import lax
from jax.experimental import pallas as pl
from jax.experimental.pallas import tpu as pltpu
from jax.experimental.pallas import tpu_sc as plsc

N_NODES = 10000
B = 128
AVG_LOG = float(
    (np.log(np.arange(8, dtype=np.float64) + 1.0)
     * np.array([0.0, 5000.0, 10000.0, 15000.0, 10000.0, 5000.0, 3000.0, 2000.0])).sum()
    / 50000.0)

_NC, _NS = 2, 16
_NW = _NC * _NS
_NPAD = 10240
_NPT = _NPAD // _NW
_E = 320000
_CHUNK = 1600
_NCHUNK = _E // _CHUNK
_G = 64
_ACC_ROWS = _NPT + 1
_NEG = -3.0e38
_POS = 3.0e38


def _make_sc_agg(fp):
    acc_n = _ACC_ROWS * fp
    mesh = plsc.VectorSubcoreMesh(core_axis_name="c", subcore_axis_name="s")
    out_type = ([jax.ShapeDtypeStruct((_NPAD * fp,), jnp.float32)] * 4
                + [jax.ShapeDtypeStruct((_NPAD,), jnp.float32)])
    scratch = [
        pltpu.VMEM((2 * _CHUNK,), jnp.int32),
        pltpu.VMEM((2 * _CHUNK,), jnp.int32),
        pltpu.VMEM((_CHUNK + _G,), jnp.int32),
        pltpu.VMEM((_CHUNK + _G,), jnp.int32),
        pltpu.VMEM((2, _G, 128), jnp.float32),
        pltpu.VMEM((acc_n,), jnp.float32),
        pltpu.VMEM((acc_n,), jnp.float32),
        pltpu.VMEM((acc_n,), jnp.float32),
        pltpu.VMEM((acc_n,), jnp.float32),
        pltpu.VMEM((_NPT + 16,), jnp.float32),
        pltpu.SemaphoreType.DMA,
        pltpu.SemaphoreType.DMA,
        pltpu.SemaphoreType.DMA,
        pltpu.SemaphoreType.DMA,
    ]

    @functools.partial(pl.kernel, out_type=out_type, mesh=mesh,
                       scratch_types=scratch,
                       compiler_params=pltpu.CompilerParams(
                           needs_layout_passes=False))
    def body(b_hbm, src_hbm, dst_hbm, s1_hbm, s2_hbm, mx_hbm, mn_hbm, cnt_hbm,
             srcb, dstb, msrc, mldst, gbuf, a1, a2, amx, amn, acnt,
             es0, es1, gs0, gs1):
        wid = lax.axis_index("s") * _NC + lax.axis_index("c")
        base = wid * _NPT
        zf = jnp.zeros((16,), jnp.float32)
        neg = jnp.full((16,), _NEG, jnp.float32)
        pos16 = jnp.full((16,), _POS, jnp.float32)
        one0 = jnp.where(lax.iota(jnp.int32, 16) == 0, 1.0, 0.0)

        def init(i, _):
            a1[pl.ds(i * 16, 16)] = zf
            a2[pl.ds(i * 16, 16)] = zf
            amx[pl.ds(i * 16, 16)] = neg
            amn[pl.ds(i * 16, 16)] = pos16
            return 0
        lax.fori_loop(0, acc_n // 16, init, 0)

        def initc(i, _):
            acnt[pl.ds(i * 16, 16)] = zf
            return 0
        lax.fori_loop(0, (_NPT + 16) // 16, initc, 0)

        def echunk_start(ci, slot):
            off = slot * _CHUNK
            pltpu.async_copy(src_hbm.at[pl.ds(ci * _CHUNK, _CHUNK)],
                             srcb.at[pl.ds(off, _CHUNK)], es0)
            pltpu.async_copy(dst_hbm.at[pl.ds(ci * _CHUNK, _CHUNK)],
                             dstb.at[pl.ds(off, _CHUNK)], es1)

        def echunk_wait(ci, slot):
            off = slot * _CHUNK
            pltpu.make_async_copy(src_hbm.at[pl.ds(ci * _CHUNK, _CHUNK)],
                                  srcb.at[pl.ds(off, _CHUNK)], es0).wait()
            pltpu.make_async_copy(dst_hbm.at[pl.ds(ci * _CHUNK, _CHUNK)],
                                  dstb.at[pl.ds(off, _CHUNK)], es1).wait()

        echunk_start(0, 0)

        def chunk_body(ci, _):
            slot = lax.rem(ci, 2)
            coff = slot * _CHUNK
            echunk_wait(ci, slot)

            @pl.when(ci + 1 < _NCHUNK)
            def _():
                echunk_start(ci + 1, 1 - slot)

            def scan(g, m):
                d = dstb[pl.ds(coff + g * 16, 16)]
                s = srcb[pl.ds(coff + g * 16, 16)]
                msk = (d >= base) & (d < base + _NPT)
                mi = jnp.where(msk, 1, 0)
                cs = plsc.cumsum(mi)
                pos = (m + cs) - mi
                plsc.store_scatter(msrc, [pos], s, mask=msk)
                plsc.store_scatter(mldst, [pos], d - base, mask=msk)
                return m + cs[15]
            m = lax.fori_loop(0, _CHUNK // 16, scan, 0)

            zi = jnp.zeros((16,), jnp.int32)

            def pad(i, _):
                msrc[pl.ds(m + i * 16, 16)] = zi
                return 0
            lax.fori_loop(0, _G // 16, pad, 0)

            nb = (m + _G - 1) // _G

            def g_start0(bi):
                pltpu.async_copy(b_hbm.at[msrc.at[pl.ds(bi * _G, _G)]],
                                 gbuf.at[0], gs0)

            def g_start1(bi):
                pltpu.async_copy(b_hbm.at[msrc.at[pl.ds(bi * _G, _G)]],
                                 gbuf.at[1], gs1)

            def g_wait0(bi):
                pltpu.make_async_copy(b_hbm.at[msrc.at[pl.ds(bi * _G, _G)]],
                                      gbuf.at[0], gs0).wait()

            def g_wait1(bi):
                pltpu.make_async_copy(b_hbm.at[msrc.at[pl.ds(bi * _G, _G)]],
                                      gbuf.at[1], gs1).wait()

            def rows(bi, s):
                nr = jnp.minimum(m - bi * _G, _G)

                def row(r, _):
                    ld = mldst[pl.ds(bi * _G + r, 16)][0]
                    off = ld * fp
                    plsc.addupdate(acnt.at[pl.ds(ld, 16)], one0)
                    for c in range(fp // 16):
                        g = gbuf[s, r, pl.ds(c * 16, 16)]
                        o = off + c * 16
                        plsc.addupdate(a1.at[pl.ds(o, 16)], g)
                        plsc.addupdate(a2.at[pl.ds(o, 16)], g * g)
                        amx[pl.ds(o, 16)] = jnp.maximum(amx[pl.ds(o, 16)], g)
                        amn[pl.ds(o, 16)] = jnp.minimum(amn[pl.ds(o, 16)], g)
                    return 0
                lax.fori_loop(0, nr, row, 0)

            @pl.when(nb > 0)
            def _():
                g_start0(0)

            def pairs(j, _):
                b0 = 2 * j

                @pl.when(b0 + 1 < nb)
                def _():
                    g_start1(b0 + 1)
                g_wait0(b0)
                rows(b0, 0)

                @pl.when(b0 + 2 < nb)
                def _():
                    g_start0(b0 + 2)

                @pl.when(b0 + 1 < nb)
                def _():
                    g_wait1(b0 + 1)
                    rows(b0 + 1, 1)
                return 0
            lax.fori_loop(0, (nb + 1) // 2, pairs, 0)
            return 0
        lax.fori_loop(0, _NCHUNK, chunk_body, 0)

        pltpu.sync_copy(a1.at[pl.ds(0, _NPT * fp)],
                        s1_hbm.at[pl.ds(base * fp, _NPT * fp)])
        pltpu.sync_copy(a2.at[pl.ds(0, _NPT * fp)],
                        s2_hbm.at[pl.ds(base * fp, _NPT * fp)])
        pltpu.sync_copy(amx.at[pl.ds(0, _NPT * fp)],
                        mx_hbm.at[pl.ds(base * fp, _NPT * fp)])
        pltpu.sync_copy(amn.at[pl.ds(0, _NPT * fp)],
                        mn_hbm.at[pl.ds(base * fp, _NPT * fp)])
        pltpu.sync_copy(acnt.at[pl.ds(0, _NPT)], cnt_hbm.at[pl.ds(base, _NPT)])

    return body


_sc_agg = {80: _make_sc_agg(80), 64: _make_sc_agg(64)}


def _combine(x, a, s1, s2, mx, mn, cnt, deg, p):
    f = x.shape[1]
    c = cnt[:, None]
    d = deg[:, None]
    mean = (c * a + s1) / d
    mean_sq = (c * a * a + 2.0 * a * s1 + s2) / d
    std = jnp.sqrt(jnp.maximum(mean_sq - mean * mean, 0.0) + 1e-5)
    mxo = jnp.where(c > 0, a + mx, 0.0)
    mno = jnp.where(c > 0, a + mn, 0.0)
    agg = jnp.concatenate([mean, mxo, mno, std], axis=-1)
    logd = jnp.log(deg + 1.0)[:, None]
    w = p['post_W']
    out = (x @ w[:f] + agg @ w[f:f + 4 * f]
           + (logd / AVG_LOG) * (agg @ w[f + 4 * f:f + 8 * f])
           + (AVG_LOG / logd) * (agg @ w[f + 8 * f:])
           + p['post_b'])
    out = out @ p['lin_W'] + p['lin_b']
    return jax.nn.relu(out / np.sqrt(1.0 + 1e-5) * p['bn_g'] + p['bn_b'])


def _pna_layer(x, src, dst, cnt, deg, p, sc_out=None):
    f = x.shape[1]
    fp = 80 if f == 78 else 64
    a = x @ p['pre_W'][:f] + p['pre_b']
    b = x @ p['pre_W'][f:]
    bpad = jnp.zeros((_NPAD, 128), jnp.float32).at[:N_NODES, :f].set(b)
    s1f, s2f, mxf, mnf, cntf = _sc_agg[fp](bpad, src, dst)
    s1 = s1f.reshape(_NPAD, fp)[:N_NODES, :f]
    s2 = s2f.reshape(_NPAD, fp)[:N_NODES, :f]
    mx = mxf.reshape(_NPAD, fp)[:N_NODES, :f]
    mn = mnf.reshape(_NPAD, fp)[:N_NODES, :f]
    if cnt is None:
        cnt = cntf[:N_NODES]
        deg = jnp.maximum(cnt, 1.0)
    return _combine(x, a, s1, s2, mx, mn, cnt, deg, p), cnt, deg


_NB_POOL = _NPAD // 1024


def _pool_body(batch_ref, h_ref, pool_ref, gcnt_ref):
    i = pl.program_id(0)

    @pl.when(i == 0)
    def _():
        pool_ref[...] = jnp.zeros_like(pool_ref)
        gcnt_ref[...] = jnp.zeros_like(gcnt_ref)

    iot = jax.lax.broadcasted_iota(jnp.int32, (1024, B), 1)
    onehot = (batch_ref[...] == iot).astype(jnp.float32)
    pool_ref[...] += jnp.dot(onehot.T, h_ref[...],
                             preferred_element_type=jnp.float32)
    gcnt_ref[...] += jnp.sum(onehot, axis=0)[None, :]


def _pool(batch_b2, h_pad):
    return pl.pallas_call(
        _pool_body,
        grid=(_NB_POOL,),
        in_specs=[pl.BlockSpec((1024, B), lambda i: (i, 0)),
                  pl.BlockSpec((1024, 64), lambda i: (i, 0))],
        out_specs=[pl.BlockSpec((B, 64), lambda i: (0, 0)),
                   pl.BlockSpec((1, B), lambda i: (0, 0))],
        out_shape=[jax.ShapeDtypeStruct((B, 64), jnp.float32),
                   jax.ShapeDtypeStruct((1, B), jnp.float32)],
    )(batch_b2, h_pad)


def _mlp_body(pool_ref, gcnt_ref, xt_ref, wg, bg, w1a, w1b, b1, w2, b2,
              w3, b3, w4, b4, out_ref):
    gc = jnp.maximum(gcnt_ref[...].reshape(B, 1), 1.0)
    xg = pool_ref[...] / gc
    xg = jax.nn.relu(jnp.dot(xg, wg[...],
                             preferred_element_type=jnp.float32) + bg[...])
    h = jax.nn.relu(jnp.dot(xg, w1a[...], preferred_element_type=jnp.float32)
                    + jnp.dot(xt_ref[...], w1b[...],
                              preferred_element_type=jnp.float32) + b1[...])
    h = jax.nn.relu(jnp.dot(h, w2[...],
                            preferred_element_type=jnp.float32) + b2[...])
    h = jax.nn.relu(jnp.dot(h, w3[...],
                            preferred_element_type=jnp.float32) + b3[...])
    out_ref[...] = jnp.dot(h, w4[...],
                           preferred_element_type=jnp.float32) + b4[...]


def _mlp_head(pool, gcnt, xt, params):
    return pl.pallas_call(
        _mlp_body,
        out_shape=jax.ShapeDtypeStruct((B, 1), jnp.float32),
    )(pool, gcnt, xt,
      params['fc1_xd_W'], params['fc1_xd_b'][None, :],
      params['fc1_W'][:128], params['fc1_W'][128:], params['fc1_b'][None, :],
      params['fc2_W'], params['fc2_b'][None, :],
      params['fc3_W'], params['fc3_b'][None, :],
      params['out_W'], params['out_b'][None, :])


def _conv1d(h, w, b):
    o = jax.lax.conv_general_dilated(h, w, (1,), 'VALID',
                                     dimension_numbers=('NCH', 'OIH', 'NCH'))
    return o + b[None, :, None]


def kernel(x, edge_index, batch, target, params):
    src, dst = edge_index[0], edge_index[1]
    h, cnt, deg = _pna_layer(x, src, dst, None, None, params['conv1'])
    h, _, _ = _pna_layer(h, src, dst, cnt, deg, params['conv2'])
    h, _, _ = _pna_layer(h, src, dst, cnt, deg, params['conv3'])

    batch_pad = jnp.full((_NPAD,), 999, jnp.int32).at[:N_NODES].set(batch)
    h_pad = jnp.zeros((_NPAD, 64), jnp.float32).at[:N_NODES].set(h)
    batch_b2 = jnp.broadcast_to(batch_pad[:, None], (_NPAD, B))
    pool, gcnt = _pool(batch_b2, h_pad)

    e = jnp.transpose(params['emb'][target], (0, 2, 1))
    e = jax.nn.relu(_conv1d(e, params['c1_W'], params['c1_b']))
    e = jax.nn.relu(_conv1d(e, params['c2_W'], params['c2_b']))
    e = jax.nn.relu(_conv1d(e, params['c3_W'], params['c3_b']))
    xt = jnp.max(e, axis=2) @ params['pfc_W'] + params['pfc_b']
    return _mlp_head(pool, gcnt, xt, params)


# speedup vs baseline: 6.5004x; 6.0835x over previous
import functools

import jax
import jax.numpy as jnp
import numpy as np
from jax import lax
from jax.experimental import pallas as pl
from jax.experimental.pallas import tpu as pltpu
from jax.experimental.pallas import tpu_sc as plsc

N_NODES = 10000
B = 128
AVG_LOG = float(
    (np.log(np.arange(8, dtype=np.float64) + 1.0)
     * np.array([0.0, 5000.0, 10000.0, 15000.0, 10000.0, 5000.0, 3000.0, 2000.0])).sum()
    / 50000.0)

_NC, _NS = 2, 16
_NW = _NC * _NS
_NPAD = 10240
_NPT = _NPAD // _NW
_E = 320000
_C = 8000
_NCH = _E // _C
_G = 64
_ACC_ROWS = _NPT + 1
_LCAP = _E + _C + 64
_NEG = -3.0e38
_POS = 3.0e38
_DUMP = _NPT


def _prep_kernel():
    mesh = plsc.VectorSubcoreMesh(core_axis_name="c", subcore_axis_name="s")
    out_type = [
        jax.ShapeDtypeStruct((_NW * _LCAP,), jnp.int32),
        jax.ShapeDtypeStruct((_NW * _LCAP,), jnp.int32),
        jax.ShapeDtypeStruct((_NW * 16,), jnp.int32),
        jax.ShapeDtypeStruct((_NPAD,), jnp.float32),
    ]
    scratch = [
        pltpu.VMEM((2 * _C,), jnp.int32),
        pltpu.VMEM((2 * _C,), jnp.int32),
        pltpu.VMEM((_C + 16,), jnp.int32),
        pltpu.VMEM((_C + 16,), jnp.int32),
        pltpu.VMEM((_NPT + 16,), jnp.float32),
        pltpu.VMEM((16,), jnp.int32),
        pltpu.SemaphoreType.DMA,
        pltpu.SemaphoreType.DMA,
        pltpu.SemaphoreType.DMA,
        pltpu.SemaphoreType.DMA,
    ]

    @functools.partial(pl.kernel, out_type=out_type, mesh=mesh,
                       scratch_types=scratch,
                       compiler_params=pltpu.CompilerParams(
                           needs_layout_passes=False))
    def body(src_hbm, dst_hbm, ms_hbm, ml_hbm, mc_hbm, cnt_hbm,
             srcb, dstb, msrc, mldst, acnt, cstage, es0, es1, fs0, fs1):
        wid = lax.axis_index("s") * _NC + lax.axis_index("c")
        base = wid * _NPT
        zf = jnp.zeros((16,), jnp.float32)
        ones16 = jnp.full((16,), 1.0, jnp.float32)

        def initc(i, _):
            acnt[pl.ds(i * 16, 16)] = zf
            return 0
        lax.fori_loop(0, (_NPT + 16) // 16, initc, 0)

        zi = jnp.zeros((16,), jnp.int32)
        dv = jnp.full((16,), _DUMP, jnp.int32)

        def initm(i, _):
            msrc[pl.ds(i * 16, 16)] = zi
            mldst[pl.ds(i * 16, 16)] = dv
            return 0
        lax.fori_loop(0, (_C + 16) // 16, initm, 0)

        def echunk_start(ci, slot):
            off = slot * _C
            pltpu.async_copy(src_hbm.at[pl.ds(ci * _C, _C)],
                             srcb.at[pl.ds(off, _C)], es0)
            pltpu.async_copy(dst_hbm.at[pl.ds(ci * _C, _C)],
                             dstb.at[pl.ds(off, _C)], es1)

        def echunk_wait(ci, slot):
            off = slot * _C
            pltpu.make_async_copy(src_hbm.at[pl.ds(ci * _C, _C)],
                                  srcb.at[pl.ds(off, _C)], es0).wait()
            pltpu.make_async_copy(dst_hbm.at[pl.ds(ci * _C, _C)],
                                  dstb.at[pl.ds(off, _C)], es1).wait()

        echunk_start(0, 0)

        def chunk_body(ci, mtot):
            slot = lax.rem(ci, 2)
            coff = slot * _C
            echunk_wait(ci, slot)

            @pl.when(ci + 1 < _NCH)
            def _():
                echunk_start(ci + 1, 1 - slot)

            @pl.when(ci > 0)
            def _():
                pltpu.make_async_copy(
                    msrc.at[pl.ds(0, _C)],
                    ms_hbm.at[pl.ds(0, _C)], fs0).wait()
                pltpu.make_async_copy(
                    mldst.at[pl.ds(0, _C)],
                    ml_hbm.at[pl.ds(0, _C)], fs1).wait()

            def scan(g, m):
                d = dstb[pl.ds(coff + g * 16, 16)]
                s = srcb[pl.ds(coff + g * 16, 16)]
                msk = (d >= base) & (d < base + _NPT)
                mi = jnp.where(msk, 1, 0)
                cs = plsc.cumsum(mi)
                pos = (m + cs) - mi
                ld = d - base
                plsc.store_scatter(msrc, [pos], s, mask=msk)
                plsc.store_scatter(mldst, [pos], ld, mask=msk)
                plsc.addupdate_scatter(
                    acnt, [jnp.where(msk, ld, _DUMP)], ones16, mask=msk)
                return m + cs[15]
            m = lax.fori_loop(0, _C // 16, scan, 0)

            msrc[pl.ds(m, 16)] = zi
            mldst[pl.ds(m, 16)] = dv
            mpad = ((m + 7) // 8) * 8

            moff = wid * _LCAP + pl.multiple_of(mtot, 8)
            pltpu.async_copy(msrc.at[pl.ds(0, _C)],
                             ms_hbm.at[pl.ds(moff, _C)], fs0)
            pltpu.async_copy(mldst.at[pl.ds(0, _C)],
                             ml_hbm.at[pl.ds(moff, _C)], fs1)
            return mtot + mpad
        mtot = lax.fori_loop(0, _NCH, chunk_body, 0)
        pltpu.make_async_copy(msrc.at[pl.ds(0, _C)],
                              ms_hbm.at[pl.ds(0, _C)], fs0).wait()
        pltpu.make_async_copy(mldst.at[pl.ds(0, _C)],
                              ml_hbm.at[pl.ds(0, _C)], fs1).wait()

        cstage[pl.ds(0, 16)] = jnp.full((16,), 1, jnp.int32) * mtot
        pltpu.sync_copy(cstage, mc_hbm.at[pl.ds(wid * 16, 16)])
        pltpu.sync_copy(acnt.at[pl.ds(0, _NPT)], cnt_hbm.at[pl.ds(base, _NPT)])

    return body


def _layer_kernel(fp):
    acc_n = _ACC_ROWS * fp
    mesh = plsc.VectorSubcoreMesh(core_axis_name="c", subcore_axis_name="s")
    out_type = [jax.ShapeDtypeStruct((_NPAD * fp,), jnp.float32)] * 4
    scratch = [
        pltpu.VMEM((2, _G), jnp.int32),
        pltpu.VMEM((2, _G + 16), jnp.int32),
        pltpu.VMEM((2, _G, 128), jnp.float32),
        pltpu.VMEM((16,), jnp.int32),
        pltpu.VMEM((acc_n,), jnp.float32),
        pltpu.VMEM((acc_n,), jnp.float32),
        pltpu.VMEM((acc_n,), jnp.float32),
        pltpu.VMEM((acc_n,), jnp.float32),
        pltpu.SemaphoreType.DMA,
        pltpu.SemaphoreType.DMA,
        pltpu.SemaphoreType.DMA,
        pltpu.SemaphoreType.DMA,
        pltpu.SemaphoreType.DMA,
        pltpu.SemaphoreType.DMA,
    ]

    @functools.partial(pl.kernel, out_type=out_type, mesh=mesh,
                       scratch_types=scratch,
                       compiler_params=pltpu.CompilerParams(
                           needs_layout_passes=False))
    def body(b_hbm, ms_hbm, ml_hbm, mc_hbm, s1_hbm, s2_hbm, mx_hbm, mn_hbm,
             ib, lb, gbuf, cstage, a1, a2, amx, amn,
             is0, is1, ls0, ls1, gs0, gs1):
        wid = lax.axis_index("s") * _NC + lax.axis_index("c")
        base = wid * _NPT
        zf = jnp.zeros((16,), jnp.float32)
        neg = jnp.full((16,), _NEG, jnp.float32)
        pos16 = jnp.full((16,), _POS, jnp.float32)

        def init(i, _):
            a1[pl.ds(i * 16, 16)] = zf
            a2[pl.ds(i * 16, 16)] = zf
            amx[pl.ds(i * 16, 16)] = neg
            amn[pl.ds(i * 16, 16)] = pos16
            return 0
        lax.fori_loop(0, acc_n // 16, init, 0)

        pltpu.sync_copy(mc_hbm.at[pl.ds(wid * 16, 16)], cstage)
        mt = cstage[pl.ds(0, 16)][0]
        nb = (mt + _G - 1) // _G

        def i_start0(bi):
            pltpu.async_copy(ms_hbm.at[pl.ds(wid * _LCAP + bi * _G, _G)], ib.at[0], is0)
            pltpu.async_copy(ml_hbm.at[pl.ds(wid * _LCAP + bi * _G, _G)],
                             lb.at[0, pl.ds(0, _G)], ls0)

        def i_start1(bi):
            pltpu.async_copy(ms_hbm.at[pl.ds(wid * _LCAP + bi * _G, _G)], ib.at[1], is1)
            pltpu.async_copy(ml_hbm.at[pl.ds(wid * _LCAP + bi * _G, _G)],
                             lb.at[1, pl.ds(0, _G)], ls1)

        def i_wait0(bi):
            pltpu.make_async_copy(ms_hbm.at[pl.ds(wid * _LCAP + bi * _G, _G)],
                                  ib.at[0], is0).wait()
            pltpu.make_async_copy(ml_hbm.at[pl.ds(wid * _LCAP + bi * _G, _G)],
                                  lb.at[0, pl.ds(0, _G)], ls0).wait()

        def i_wait1(bi):
            pltpu.make_async_copy(ms_hbm.at[pl.ds(wid * _LCAP + bi * _G, _G)],
                                  ib.at[1], is1).wait()
            pltpu.make_async_copy(ml_hbm.at[pl.ds(wid * _LCAP + bi * _G, _G)],
                                  lb.at[1, pl.ds(0, _G)], ls1).wait()

        def g_start0():
            pltpu.async_copy(b_hbm.at[ib.at[0]], gbuf.at[0], gs0)

        def g_start1():
            pltpu.async_copy(b_hbm.at[ib.at[1]], gbuf.at[1], gs1)

        def g_wait0():
            pltpu.make_async_copy(b_hbm.at[ib.at[0]], gbuf.at[0], gs0).wait()

        def g_wait1():
            pltpu.make_async_copy(b_hbm.at[ib.at[1]], gbuf.at[1], gs1).wait()

        def rows(bi, s):
            nr = jnp.minimum(mt - bi * _G, _G)

            def row(r, _):
                ld = lb[s, pl.ds(r, 16)][0]
                off = ld * fp
                for c in range(fp // 16):
                    g = gbuf[s, r, pl.ds(c * 16, 16)]
                    o = off + c * 16
                    plsc.addupdate(a1.at[pl.ds(o, 16)], g)
                    plsc.addupdate(a2.at[pl.ds(o, 16)], g * g)
                    amx[pl.ds(o, 16)] = jnp.maximum(amx[pl.ds(o, 16)], g)
                    amn[pl.ds(o, 16)] = jnp.minimum(amn[pl.ds(o, 16)], g)
                return 0
            lax.fori_loop(0, nr, row, 0)

        @pl.when(nb > 0)
        def _():
            i_start0(0)
            i_wait0(0)
            g_start0()

            @pl.when(nb > 1)
            def _():
                i_start1(1)

        def pairs(j, _):
            b0 = 2 * j

            @pl.when(b0 + 1 < nb)
            def _():
                i_wait1(b0 + 1)
                g_start1()
            g_wait0()
            rows(b0, 0)

            @pl.when(b0 + 2 < nb)
            def _():
                i_start0(b0 + 2)

            @pl.when(b0 + 1 < nb)
            def _():
                g_wait1()
                rows(b0 + 1, 1)

            @pl.when(b0 + 3 < nb)
            def _():
                i_start1(b0 + 3)

            @pl.when(b0 + 2 < nb)
            def _():
                i_wait0(b0 + 2)
                g_start0()
            return 0
        lax.fori_loop(0, (nb + 1) // 2, pairs, 0)

        pltpu.sync_copy(a1.at[pl.ds(0, _NPT * fp)],
                        s1_hbm.at[pl.ds(base * fp, _NPT * fp)])
        pltpu.sync_copy(a2.at[pl.ds(0, _NPT * fp)],
                        s2_hbm.at[pl.ds(base * fp, _NPT * fp)])
        pltpu.sync_copy(amx.at[pl.ds(0, _NPT * fp)],
                        mx_hbm.at[pl.ds(base * fp, _NPT * fp)])
        pltpu.sync_copy(amn.at[pl.ds(0, _NPT * fp)],
                        mn_hbm.at[pl.ds(base * fp, _NPT * fp)])

    return body


_prep = _prep_kernel()
_layer = {80: _layer_kernel(80), 64: _layer_kernel(64)}


def _combine(x, a, s1, s2, mx, mn, cnt, deg, p):
    f = x.shape[1]
    c = cnt[:, None]
    d = deg[:, None]
    mean = (c * a + s1) / d
    mean_sq = (c * a * a + 2.0 * a * s1 + s2) / d
    std = jnp.sqrt(jnp.maximum(mean_sq - mean * mean, 0.0) + 1e-5)
    mxo = jnp.where(c > 0, a + mx, 0.0)
    mno = jnp.where(c > 0, a + mn, 0.0)
    agg = jnp.concatenate([mean, mxo, mno, std], axis=-1)
    logd = jnp.log(deg + 1.0)[:, None]
    w = p['post_W']
    out = (x @ w[:f] + agg @ w[f:f + 4 * f]
           + (logd / AVG_LOG) * (agg @ w[f + 4 * f:f + 8 * f])
           + (AVG_LOG / logd) * (agg @ w[f + 8 * f:])
           + p['post_b'])
    out = out @ p['lin_W'] + p['lin_b']
    return jax.nn.relu(out / np.sqrt(1.0 + 1e-5) * p['bn_g'] + p['bn_b'])


def _pna_layer(x, msl, mll, mcl, cnt, deg, p):
    f = x.shape[1]
    fp = 80 if f == 78 else 64
    a = x @ p['pre_W'][:f] + p['pre_b']
    b = x @ p['pre_W'][f:]
    bpad = jnp.zeros((_NPAD, 128), jnp.float32).at[:N_NODES, :f].set(b)
    s1f, s2f, mxf, mnf = _layer[fp](bpad, msl, mll, mcl)
    s1 = s1f.reshape(_NPAD, fp)[:N_NODES, :f]
    s2 = s2f.reshape(_NPAD, fp)[:N_NODES, :f]
    mx = mxf.reshape(_NPAD, fp)[:N_NODES, :f]
    mn = mnf.reshape(_NPAD, fp)[:N_NODES, :f]
    return _combine(x, a, s1, s2, mx, mn, cnt, deg, p)


_NB_POOL = _NPAD // 1024


def _pool_body(batch_ref, h_ref, pool_ref, gcnt_ref):
    i = pl.program_id(0)

    @pl.when(i == 0)
    def _():
        pool_ref[...] = jnp.zeros_like(pool_ref)
        gcnt_ref[...] = jnp.zeros_like(gcnt_ref)

    iot = jax.lax.broadcasted_iota(jnp.int32, (1024, B), 1)
    onehot = (batch_ref[...] == iot).astype(jnp.float32)
    pool_ref[...] += jnp.dot(onehot.T, h_ref[...],
                             preferred_element_type=jnp.float32)
    gcnt_ref[...] += jnp.sum(onehot, axis=0)[None, :]


def _pool(batch_b2, h_pad):
    return pl.pallas_call(
        _pool_body,
        grid=(_NB_POOL,),
        in_specs=[pl.BlockSpec((1024, B), lambda i: (i, 0)),
                  pl.BlockSpec((1024, 64), lambda i: (i, 0))],
        out_specs=[pl.BlockSpec((B, 64), lambda i: (0, 0)),
                   pl.BlockSpec((1, B), lambda i: (0, 0))],
        out_shape=[jax.ShapeDtypeStruct((B, 64), jnp.float32),
                   jax.ShapeDtypeStruct((1, B), jnp.float32)],
    )(batch_b2, h_pad)


def _mlp_body(pool_ref, gcnt_ref, xt_ref, wg, bg, w1a, w1b, b1, w2, b2,
              w3, b3, w4, b4, out_ref):
    gc = jnp.maximum(gcnt_ref[...].reshape(B, 1), 1.0)
    xg = pool_ref[...] / gc
    xg = jax.nn.relu(jnp.dot(xg, wg[...],
                             preferred_element_type=jnp.float32) + bg[...])
    h = jax.nn.relu(jnp.dot(xg, w1a[...], preferred_element_type=jnp.float32)
                    + jnp.dot(xt_ref[...], w1b[...],
                              preferred_element_type=jnp.float32) + b1[...])
    h = jax.nn.relu(jnp.dot(h, w2[...],
                            preferred_element_type=jnp.float32) + b2[...])
    h = jax.nn.relu(jnp.dot(h, w3[...],
                            preferred_element_type=jnp.float32) + b3[...])
    out_ref[...] = jnp.dot(h, w4[...],
                           preferred_element_type=jnp.float32) + b4[...]


def _mlp_head(pool, gcnt, xt, params):
    return pl.pallas_call(
        _mlp_body,
        out_shape=jax.ShapeDtypeStruct((B, 1), jnp.float32),
    )(pool, gcnt, xt,
      params['fc1_xd_W'], params['fc1_xd_b'][None, :],
      params['fc1_W'][:128], params['fc1_W'][128:], params['fc1_b'][None, :],
      params['fc2_W'], params['fc2_b'][None, :],
      params['fc3_W'], params['fc3_b'][None, :],
      params['out_W'], params['out_b'][None, :])


def _conv1d(h, w, b):
    o = jax.lax.conv_general_dilated(h, w, (1,), 'VALID',
                                     dimension_numbers=('NCH', 'OIH', 'NCH'))
    return o + b[None, :, None]


def kernel(x, edge_index, batch, target, params):
    src, dst = edge_index[0], edge_index[1]
    msl, mll, mcl, cntf = _prep(src, dst)
    cnt = cntf[:N_NODES]
    deg = jnp.maximum(cnt, 1.0)

    h = _pna_layer(x, msl, mll, mcl, cnt, deg, params['conv1'])
    h = _pna_layer(h, msl, mll, mcl, cnt, deg, params['conv2'])
    h = _pna_layer(h, msl, mll, mcl, cnt, deg, params['conv3'])

    batch_pad = jnp.full((_NPAD,), 999, jnp.int32).at[:N_NODES].set(batch)
    h_pad = jnp.zeros((_NPAD, 64), jnp.float32).at[:N_NODES].set(h)
    batch_b2 = jnp.broadcast_to(batch_pad[:, None], (_NPAD, B))
    pool, gcnt = _pool(batch_b2, h_pad)

    e = jnp.transpose(params['emb'][target], (0, 2, 1))
    e = jax.nn.relu(_conv1d(e, params['c1_W'], params['c1_b']))
    e = jax.nn.relu(_conv1d(e, params['c2_W'], params['c2_b']))
    e = jax.nn.relu(_conv1d(e, params['c3_W'], params['c3_b']))
    xt = jnp.max(e, axis=2) @ params['pfc_W'] + params['pfc_b']
    return _mlp_head(pool, gcnt, xt, params)


# speedup vs baseline: 7.9484x; 1.2227x over previous
import functools

import jax
import jax.numpy as jnp
import numpy as np
from jax import lax
from jax.experimental import pallas as pl
from jax.experimental.pallas import tpu as pltpu
from jax.experimental.pallas import tpu_sc as plsc

N_NODES = 10000
B = 128
AVG_LOG = float(
    (np.log(np.arange(8, dtype=np.float64) + 1.0)
     * np.array([0.0, 5000.0, 10000.0, 15000.0, 10000.0, 5000.0, 3000.0, 2000.0])).sum()
    / 50000.0)

_NC, _NS = 2, 16
_NW = _NC * _NS
_NPAD = 10240
_NPT = _NPAD // _NW
_E = 320000
_C = 8000
_NCH = _E // _C
_G = 64
_ACC_ROWS = _NPT + 1
_LCAP = _E + _C + 64
_NEG = -3.0e38
_POS = 3.0e38
_DUMP = _NPT


def _prep_kernel():
    mesh = plsc.VectorSubcoreMesh(core_axis_name="c", subcore_axis_name="s")
    out_type = [
        jax.ShapeDtypeStruct((_NW * _LCAP,), jnp.int32),
        jax.ShapeDtypeStruct((_NW * _LCAP,), jnp.int32),
        jax.ShapeDtypeStruct((_NW * 16,), jnp.int32),
        jax.ShapeDtypeStruct((_NPAD,), jnp.float32),
    ]
    scratch = [
        pltpu.VMEM((2 * _C,), jnp.int32),
        pltpu.VMEM((2 * _C,), jnp.int32),
        pltpu.VMEM((_C + 16,), jnp.int32),
        pltpu.VMEM((_C + 16,), jnp.int32),
        pltpu.VMEM((_NPT + 16,), jnp.float32),
        pltpu.VMEM((16,), jnp.int32),
        pltpu.SemaphoreType.DMA,
        pltpu.SemaphoreType.DMA,
        pltpu.SemaphoreType.DMA,
        pltpu.SemaphoreType.DMA,
    ]

    @functools.partial(pl.kernel, out_type=out_type, mesh=mesh,
                       scratch_types=scratch,
                       compiler_params=pltpu.CompilerParams(
                           needs_layout_passes=False))
    def body(src_hbm, dst_hbm, ms_hbm, ml_hbm, mc_hbm, cnt_hbm,
             srcb, dstb, msrc, mldst, acnt, cstage, es0, es1, fs0, fs1):
        wid = lax.axis_index("s") * _NC + lax.axis_index("c")
        base = wid * _NPT
        zf = jnp.zeros((16,), jnp.float32)
        ones16 = jnp.full((16,), 1.0, jnp.float32)

        def initc(i, _):
            acnt[pl.ds(i * 16, 16)] = zf
            return 0
        lax.fori_loop(0, (_NPT + 16) // 16, initc, 0)

        zi = jnp.zeros((16,), jnp.int32)
        dv = jnp.full((16,), _DUMP, jnp.int32)

        def initm(i, _):
            msrc[pl.ds(i * 16, 16)] = zi
            mldst[pl.ds(i * 16, 16)] = dv
            return 0
        lax.fori_loop(0, (_C + 16) // 16, initm, 0)

        def echunk_start(ci, slot):
            off = slot * _C
            pltpu.async_copy(src_hbm.at[pl.ds(ci * _C, _C)],
                             srcb.at[pl.ds(off, _C)], es0)
            pltpu.async_copy(dst_hbm.at[pl.ds(ci * _C, _C)],
                             dstb.at[pl.ds(off, _C)], es1)

        def echunk_wait(ci, slot):
            off = slot * _C
            pltpu.make_async_copy(src_hbm.at[pl.ds(ci * _C, _C)],
                                  srcb.at[pl.ds(off, _C)], es0).wait()
            pltpu.make_async_copy(dst_hbm.at[pl.ds(ci * _C, _C)],
                                  dstb.at[pl.ds(off, _C)], es1).wait()

        echunk_start(0, 0)

        def chunk_body(ci, mtot):
            slot = lax.rem(ci, 2)
            coff = slot * _C
            echunk_wait(ci, slot)

            @pl.when(ci + 1 < _NCH)
            def _():
                echunk_start(ci + 1, 1 - slot)

            @pl.when(ci > 0)
            def _():
                pltpu.make_async_copy(
                    msrc.at[pl.ds(0, _C)],
                    ms_hbm.at[pl.ds(0, _C)], fs0).wait()
                pltpu.make_async_copy(
                    mldst.at[pl.ds(0, _C)],
                    ml_hbm.at[pl.ds(0, _C)], fs1).wait()

            def scan(g, m):
                d = dstb[pl.ds(coff + g * 16, 16)]
                s = srcb[pl.ds(coff + g * 16, 16)]
                msk = (d >= base) & (d < base + _NPT)
                mi = jnp.where(msk, 1, 0)
                cs = plsc.cumsum(mi)
                pos = (m + cs) - mi
                ld = d - base
                plsc.store_scatter(msrc, [pos], s, mask=msk)
                plsc.store_scatter(mldst, [pos], ld, mask=msk)
                plsc.addupdate_scatter(
                    acnt, [jnp.where(msk, ld, _DUMP)], ones16, mask=msk)
                return m + cs[15]
            m = lax.fori_loop(0, _C // 16, scan, 0)

            msrc[pl.ds(m, 16)] = zi
            mldst[pl.ds(m, 16)] = dv
            mpad = ((m + 7) // 8) * 8

            moff = wid * _LCAP + pl.multiple_of(mtot, 8)
            pltpu.async_copy(msrc.at[pl.ds(0, _C)],
                             ms_hbm.at[pl.ds(moff, _C)], fs0)
            pltpu.async_copy(mldst.at[pl.ds(0, _C)],
                             ml_hbm.at[pl.ds(moff, _C)], fs1)
            return mtot + mpad
        mtot = lax.fori_loop(0, _NCH, chunk_body, 0)
        pltpu.make_async_copy(msrc.at[pl.ds(0, _C)],
                              ms_hbm.at[pl.ds(0, _C)], fs0).wait()
        pltpu.make_async_copy(mldst.at[pl.ds(0, _C)],
                              ml_hbm.at[pl.ds(0, _C)], fs1).wait()

        cstage[pl.ds(0, 16)] = jnp.full((16,), 1, jnp.int32) * mtot
        pltpu.sync_copy(cstage, mc_hbm.at[pl.ds(wid * 16, 16)])
        pltpu.sync_copy(acnt.at[pl.ds(0, _NPT)], cnt_hbm.at[pl.ds(base, _NPT)])

    return body


def _layer_kernel(fp):
    acc_n = _ACC_ROWS * fp
    mesh = plsc.VectorSubcoreMesh(core_axis_name="c", subcore_axis_name="s")
    out_type = [jax.ShapeDtypeStruct((_NPAD * fp,), jnp.float32)] * 4
    scratch = [
        pltpu.VMEM((2, _G), jnp.int32),
        pltpu.VMEM((2, _G + 16), jnp.int32),
        pltpu.VMEM((2, _G, 128), jnp.float32),
        pltpu.VMEM((16,), jnp.int32),
        pltpu.VMEM((acc_n,), jnp.float32),
        pltpu.VMEM((acc_n,), jnp.float32),
        pltpu.VMEM((acc_n,), jnp.float32),
        pltpu.VMEM((acc_n,), jnp.float32),
        pltpu.SemaphoreType.DMA,
        pltpu.SemaphoreType.DMA,
        pltpu.SemaphoreType.DMA,
        pltpu.SemaphoreType.DMA,
        pltpu.SemaphoreType.DMA,
        pltpu.SemaphoreType.DMA,
    ]

    @functools.partial(pl.kernel, out_type=out_type, mesh=mesh,
                       scratch_types=scratch,
                       compiler_params=pltpu.CompilerParams(
                           needs_layout_passes=False))
    def body(b_hbm, ms_hbm, ml_hbm, mc_hbm, s1_hbm, s2_hbm, mx_hbm, mn_hbm,
             ib, lb, gbuf, cstage, a1, a2, amx, amn,
             is0, is1, ls0, ls1, gs0, gs1):
        wid = lax.axis_index("s") * _NC + lax.axis_index("c")
        base = wid * _NPT
        zf = jnp.zeros((16,), jnp.float32)
        neg = jnp.full((16,), _NEG, jnp.float32)
        pos16 = jnp.full((16,), _POS, jnp.float32)

        def init(i, _):
            a1[pl.ds(i * 16, 16)] = zf
            a2[pl.ds(i * 16, 16)] = zf
            amx[pl.ds(i * 16, 16)] = neg
            amn[pl.ds(i * 16, 16)] = pos16
            return 0
        lax.fori_loop(0, acc_n // 16, init, 0)

        pltpu.sync_copy(mc_hbm.at[pl.ds(wid * 16, 16)], cstage)
        mt = cstage[pl.ds(0, 16)][0]
        nb = (mt + _G - 1) // _G

        def i_start0(bi):
            pltpu.async_copy(ms_hbm.at[pl.ds(wid * _LCAP + bi * _G, _G)], ib.at[0], is0)
            pltpu.async_copy(ml_hbm.at[pl.ds(wid * _LCAP + bi * _G, _G)],
                             lb.at[0, pl.ds(0, _G)], ls0)

        def i_start1(bi):
            pltpu.async_copy(ms_hbm.at[pl.ds(wid * _LCAP + bi * _G, _G)], ib.at[1], is1)
            pltpu.async_copy(ml_hbm.at[pl.ds(wid * _LCAP + bi * _G, _G)],
                             lb.at[1, pl.ds(0, _G)], ls1)

        def i_wait0(bi):
            pltpu.make_async_copy(ms_hbm.at[pl.ds(wid * _LCAP + bi * _G, _G)],
                                  ib.at[0], is0).wait()
            pltpu.make_async_copy(ml_hbm.at[pl.ds(wid * _LCAP + bi * _G, _G)],
                                  lb.at[0, pl.ds(0, _G)], ls0).wait()

        def i_wait1(bi):
            pltpu.make_async_copy(ms_hbm.at[pl.ds(wid * _LCAP + bi * _G, _G)],
                                  ib.at[1], is1).wait()
            pltpu.make_async_copy(ml_hbm.at[pl.ds(wid * _LCAP + bi * _G, _G)],
                                  lb.at[1, pl.ds(0, _G)], ls1).wait()

        def g_start0():
            pltpu.async_copy(b_hbm.at[ib.at[0]], gbuf.at[0], gs0)

        def g_start1():
            pltpu.async_copy(b_hbm.at[ib.at[1]], gbuf.at[1], gs1)

        def g_wait0():
            pltpu.make_async_copy(b_hbm.at[ib.at[0]], gbuf.at[0], gs0).wait()

        def g_wait1():
            pltpu.make_async_copy(b_hbm.at[ib.at[1]], gbuf.at[1], gs1).wait()

        def rows(bi, s):
            nr = jnp.minimum(mt - bi * _G, _G)

            def row(r, ld):
                ld_next = lb[s, pl.ds(r + 1, 16)][0]
                off = ld * fp
                for c in range(fp // 16):
                    g = gbuf[s, r, pl.ds(c * 16, 16)]
                    o = off + c * 16
                    plsc.addupdate(a1.at[pl.ds(o, 16)], g)
                    plsc.addupdate(a2.at[pl.ds(o, 16)], g * g)
                    amx[pl.ds(o, 16)] = jnp.maximum(amx[pl.ds(o, 16)], g)
                    amn[pl.ds(o, 16)] = jnp.minimum(amn[pl.ds(o, 16)], g)
                return ld_next
            lax.fori_loop(0, nr, row, lb[s, pl.ds(0, 16)][0])

        @pl.when(nb > 0)
        def _():
            i_start0(0)
            i_wait0(0)
            g_start0()

            @pl.when(nb > 1)
            def _():
                i_start1(1)

        def pairs(j, _):
            b0 = 2 * j

            @pl.when(b0 + 1 < nb)
            def _():
                i_wait1(b0 + 1)
                g_start1()
            g_wait0()
            rows(b0, 0)

            @pl.when(b0 + 2 < nb)
            def _():
                i_start0(b0 + 2)

            @pl.when(b0 + 1 < nb)
            def _():
                g_wait1()
                rows(b0 + 1, 1)

            @pl.when(b0 + 3 < nb)
            def _():
                i_start1(b0 + 3)

            @pl.when(b0 + 2 < nb)
            def _():
                i_wait0(b0 + 2)
                g_start0()
            return 0
        lax.fori_loop(0, (nb + 1) // 2, pairs, 0)

        pltpu.sync_copy(a1.at[pl.ds(0, _NPT * fp)],
                        s1_hbm.at[pl.ds(base * fp, _NPT * fp)])
        pltpu.sync_copy(a2.at[pl.ds(0, _NPT * fp)],
                        s2_hbm.at[pl.ds(base * fp, _NPT * fp)])
        pltpu.sync_copy(amx.at[pl.ds(0, _NPT * fp)],
                        mx_hbm.at[pl.ds(base * fp, _NPT * fp)])
        pltpu.sync_copy(amn.at[pl.ds(0, _NPT * fp)],
                        mn_hbm.at[pl.ds(base * fp, _NPT * fp)])

    return body


_prep = _prep_kernel()
_layer = {80: _layer_kernel(80), 64: _layer_kernel(64)}


def _combine(x, a, s1, s2, mx, mn, cnt, deg, p):
    f = x.shape[1]
    c = cnt[:, None]
    d = deg[:, None]
    mean = (c * a + s1) / d
    mean_sq = (c * a * a + 2.0 * a * s1 + s2) / d
    std = jnp.sqrt(jnp.maximum(mean_sq - mean * mean, 0.0) + 1e-5)
    mxo = jnp.where(c > 0, a + mx, 0.0)
    mno = jnp.where(c > 0, a + mn, 0.0)
    agg = jnp.concatenate([mean, mxo, mno, std], axis=-1)
    logd = jnp.log(deg + 1.0)[:, None]
    w = p['post_W']
    out = (x @ w[:f] + agg @ w[f:f + 4 * f]
           + (logd / AVG_LOG) * (agg @ w[f + 4 * f:f + 8 * f])
           + (AVG_LOG / logd) * (agg @ w[f + 8 * f:])
           + p['post_b'])
    out = out @ p['lin_W'] + p['lin_b']
    return jax.nn.relu(out / np.sqrt(1.0 + 1e-5) * p['bn_g'] + p['bn_b'])


def _pna_layer(x, msl, mll, mcl, cnt, deg, p):
    f = x.shape[1]
    fp = 80 if f == 78 else 64
    a = x @ p['pre_W'][:f] + p['pre_b']
    b = x @ p['pre_W'][f:]
    bpad = jnp.zeros((_NPAD, 128), jnp.float32).at[:N_NODES, :f].set(b)
    s1f, s2f, mxf, mnf = _layer[fp](bpad, msl, mll, mcl)
    s1 = s1f.reshape(_NPAD, fp)[:N_NODES, :f]
    s2 = s2f.reshape(_NPAD, fp)[:N_NODES, :f]
    mx = mxf.reshape(_NPAD, fp)[:N_NODES, :f]
    mn = mnf.reshape(_NPAD, fp)[:N_NODES, :f]
    return _combine(x, a, s1, s2, mx, mn, cnt, deg, p)


_NB_POOL = _NPAD // 1024


def _pool_body(batch_ref, h_ref, pool_ref, gcnt_ref):
    i = pl.program_id(0)

    @pl.when(i == 0)
    def _():
        pool_ref[...] = jnp.zeros_like(pool_ref)
        gcnt_ref[...] = jnp.zeros_like(gcnt_ref)

    iot = jax.lax.broadcasted_iota(jnp.int32, (1024, B), 1)
    onehot = (batch_ref[...] == iot).astype(jnp.float32)
    pool_ref[...] += jnp.dot(onehot.T, h_ref[...],
                             preferred_element_type=jnp.float32)
    gcnt_ref[...] += jnp.sum(onehot, axis=0)[None, :]


def _pool(batch_b2, h_pad):
    return pl.pallas_call(
        _pool_body,
        grid=(_NB_POOL,),
        in_specs=[pl.BlockSpec((1024, B), lambda i: (i, 0)),
                  pl.BlockSpec((1024, 64), lambda i: (i, 0))],
        out_specs=[pl.BlockSpec((B, 64), lambda i: (0, 0)),
                   pl.BlockSpec((1, B), lambda i: (0, 0))],
        out_shape=[jax.ShapeDtypeStruct((B, 64), jnp.float32),
                   jax.ShapeDtypeStruct((1, B), jnp.float32)],
    )(batch_b2, h_pad)


def _mlp_body(pool_ref, gcnt_ref, xt_ref, wg, bg, w1a, w1b, b1, w2, b2,
              w3, b3, w4, b4, out_ref):
    gc = jnp.maximum(gcnt_ref[...].reshape(B, 1), 1.0)
    xg = pool_ref[...] / gc
    xg = jax.nn.relu(jnp.dot(xg, wg[...],
                             preferred_element_type=jnp.float32) + bg[...])
    h = jax.nn.relu(jnp.dot(xg, w1a[...], preferred_element_type=jnp.float32)
                    + jnp.dot(xt_ref[...], w1b[...],
                              preferred_element_type=jnp.float32) + b1[...])
    h = jax.nn.relu(jnp.dot(h, w2[...],
                            preferred_element_type=jnp.float32) + b2[...])
    h = jax.nn.relu(jnp.dot(h, w3[...],
                            preferred_element_type=jnp.float32) + b3[...])
    out_ref[...] = jnp.dot(h, w4[...],
                           preferred_element_type=jnp.float32) + b4[...]


def _mlp_head(pool, gcnt, xt, params):
    return pl.pallas_call(
        _mlp_body,
        out_shape=jax.ShapeDtypeStruct((B, 1), jnp.float32),
    )(pool, gcnt, xt,
      params['fc1_xd_W'], params['fc1_xd_b'][None, :],
      params['fc1_W'][:128], params['fc1_W'][128:], params['fc1_b'][None, :],
      params['fc2_W'], params['fc2_b'][None, :],
      params['fc3_W'], params['fc3_b'][None, :],
      params['out_W'], params['out_b'][None, :])


def _prot_body(tgt_ref, emb_ref, w1_ref, b1_ref, w2_ref, b2_ref,
               w3_ref, b3_ref, wp_ref, bp_ref, out_ref):
    emb = emb_ref[...]
    iot27 = jax.lax.broadcasted_iota(jnp.int32, (1000, 27), 1)
    for j in range(8):
        tok = tgt_ref[j]
        oh = (tok[:, None] == iot27).astype(jnp.float32)
        e = jnp.dot(oh, emb, preferred_element_type=jnp.float32)
        p = jnp.dot(e, w1_ref[...], preferred_element_type=jnp.float32)
        o1 = b1_ref[...]
        for k in range(8):
            o1 = o1 + p[k:k + 993, k * 32:(k + 1) * 32]
        o1 = jax.nn.relu(o1)
        x2 = jnp.concatenate([o1[k:k + 986, :] for k in range(8)], axis=1)
        o2 = jax.nn.relu(jnp.dot(x2, w2_ref[...],
                                 preferred_element_type=jnp.float32)
                         + b2_ref[...])
        x3 = jnp.concatenate([o2[k:k + 979, :] for k in range(8)], axis=1)
        o3 = jax.nn.relu(jnp.dot(x3, w3_ref[...],
                                 preferred_element_type=jnp.float32)
                         + b3_ref[...])
        mx = jnp.max(o3, axis=0, keepdims=True)
        out_ref[j, :] = (jnp.dot(mx, wp_ref[...],
                                 preferred_element_type=jnp.float32)
                         + bp_ref[...])[0]


def _protein(target, params):
    w1 = jnp.transpose(params['c1_W'], (1, 2, 0)).reshape(128, 256)
    b1 = jnp.broadcast_to(params['c1_b'][None, :], (993, 32))
    w2 = jnp.transpose(params['c2_W'], (2, 1, 0)).reshape(256, 64)
    b2 = params['c2_b'][None, :]
    w3 = jnp.transpose(params['c3_W'], (2, 1, 0)).reshape(512, 96)
    b3 = params['c3_b'][None, :]
    return pl.pallas_call(
        _prot_body,
        grid=(B // 8,),
        in_specs=[pl.BlockSpec((8, 1000), lambda i: (i, 0))]
        + [pl.BlockSpec(s, lambda i: tuple([0] * len(s)))
           for s in [(27, 128), (128, 256), (993, 32), (256, 64), (1, 64),
                     (512, 96), (1, 96), (96, 128), (1, 128)]],
        out_specs=pl.BlockSpec((8, 128), lambda i: (i, 0)),
        out_shape=jax.ShapeDtypeStruct((B, 128), jnp.float32),
    )(target, params['emb'], w1, b1, w2, b2, w3, b3,
      params['pfc_W'], params['pfc_b'][None, :])


def kernel(x, edge_index, batch, target, params):
    src, dst = edge_index[0], edge_index[1]
    msl, mll, mcl, cntf = _prep(src, dst)
    cnt = cntf[:N_NODES]
    deg = jnp.maximum(cnt, 1.0)

    h = _pna_layer(x, msl, mll, mcl, cnt, deg, params['conv1'])
    h = _pna_layer(h, msl, mll, mcl, cnt, deg, params['conv2'])
    h = _pna_layer(h, msl, mll, mcl, cnt, deg, params['conv3'])

    batch_pad = jnp.full((_NPAD,), 999, jnp.int32).at[:N_NODES].set(batch)
    h_pad = jnp.zeros((_NPAD, 64), jnp.float32).at[:N_NODES].set(h)
    batch_b2 = jnp.broadcast_to(batch_pad[:, None], (_NPAD, B))
    pool, gcnt = _pool(batch_b2, h_pad)

    xt = _protein(target, params)
    return _mlp_head(pool, gcnt, xt, params)


# speedup vs baseline: 8.4713x; 1.0658x over previous
import functools

import jax
import jax.numpy as jnp
import numpy as np
from jax import lax
from jax.experimental import pallas as pl
from jax.experimental.pallas import tpu as pltpu
from jax.experimental.pallas import tpu_sc as plsc

N_NODES = 10000
B = 128
AVG_LOG = float(
    (np.log(np.arange(8, dtype=np.float64) + 1.0)
     * np.array([0.0, 5000.0, 10000.0, 15000.0, 10000.0, 5000.0, 3000.0, 2000.0])).sum()
    / 50000.0)

_NC, _NS = 2, 16
_NW = _NC * _NS
_NPAD = 10240
_NPT = _NPAD // _NW
_E = 320000
_C = 8000
_NCH = _E // _C
_G = 64
_ACC_ROWS = _NPT + 1
_LCAP = _E + _C + 64
_NEG = -3.0e38
_POS = 3.0e38
_DUMP = _NPT


def _prep_kernel():
    mesh = plsc.VectorSubcoreMesh(core_axis_name="c", subcore_axis_name="s")
    out_type = [
        jax.ShapeDtypeStruct((_NW * _LCAP,), jnp.int32),
        jax.ShapeDtypeStruct((_NW * _LCAP,), jnp.int32),
        jax.ShapeDtypeStruct((_NW * 16,), jnp.int32),
        jax.ShapeDtypeStruct((_NPAD,), jnp.float32),
    ]
    scratch = [
        pltpu.VMEM((2 * _C,), jnp.int32),
        pltpu.VMEM((2 * _C,), jnp.int32),
        pltpu.VMEM((_C + 16,), jnp.int32),
        pltpu.VMEM((_C + 16,), jnp.int32),
        pltpu.VMEM((_NPT + 16,), jnp.float32),
        pltpu.VMEM((16,), jnp.int32),
        pltpu.SemaphoreType.DMA,
        pltpu.SemaphoreType.DMA,
        pltpu.SemaphoreType.DMA,
        pltpu.SemaphoreType.DMA,
    ]

    @functools.partial(pl.kernel, out_type=out_type, mesh=mesh,
                       scratch_types=scratch,
                       compiler_params=pltpu.CompilerParams(
                           needs_layout_passes=False))
    def body(src_hbm, dst_hbm, ms_hbm, ml_hbm, mc_hbm, cnt_hbm,
             srcb, dstb, msrc, mldst, acnt, cstage, es0, es1, fs0, fs1):
        wid = lax.axis_index("s") * _NC + lax.axis_index("c")
        base = wid * _NPT
        zf = jnp.zeros((16,), jnp.float32)
        ones16 = jnp.full((16,), 1.0, jnp.float32)

        def initc(i, _):
            acnt[pl.ds(i * 16, 16)] = zf
            return 0
        lax.fori_loop(0, (_NPT + 16) // 16, initc, 0)

        zi = jnp.zeros((16,), jnp.int32)
        dv = jnp.full((16,), _DUMP, jnp.int32)

        def initm(i, _):
            msrc[pl.ds(i * 16, 16)] = zi
            mldst[pl.ds(i * 16, 16)] = dv
            return 0
        lax.fori_loop(0, (_C + 16) // 16, initm, 0)

        def echunk_start(ci, slot):
            off = slot * _C
            pltpu.async_copy(src_hbm.at[pl.ds(ci * _C, _C)],
                             srcb.at[pl.ds(off, _C)], es0)
            pltpu.async_copy(dst_hbm.at[pl.ds(ci * _C, _C)],
                             dstb.at[pl.ds(off, _C)], es1)

        def echunk_wait(ci, slot):
            off = slot * _C
            pltpu.make_async_copy(src_hbm.at[pl.ds(ci * _C, _C)],
                                  srcb.at[pl.ds(off, _C)], es0).wait()
            pltpu.make_async_copy(dst_hbm.at[pl.ds(ci * _C, _C)],
                                  dstb.at[pl.ds(off, _C)], es1).wait()

        echunk_start(0, 0)

        def chunk_body(ci, mtot):
            slot = lax.rem(ci, 2)
            coff = slot * _C
            echunk_wait(ci, slot)

            @pl.when(ci + 1 < _NCH)
            def _():
                echunk_start(ci + 1, 1 - slot)

            @pl.when(ci > 0)
            def _():
                pltpu.make_async_copy(
                    msrc.at[pl.ds(0, _C)],
                    ms_hbm.at[pl.ds(0, _C)], fs0).wait()
                pltpu.make_async_copy(
                    mldst.at[pl.ds(0, _C)],
                    ml_hbm.at[pl.ds(0, _C)], fs1).wait()

            def scan(g, m):
                d = dstb[pl.ds(coff + g * 16, 16)]
                s = srcb[pl.ds(coff + g * 16, 16)]
                msk = (d >= base) & (d < base + _NPT)
                mi = jnp.where(msk, 1, 0)
                cs = plsc.cumsum(mi)
                pos = (m + cs) - mi
                ld = d - base
                plsc.store_scatter(msrc, [pos], s, mask=msk)
                plsc.store_scatter(mldst, [pos], ld, mask=msk)
                plsc.addupdate_scatter(
                    acnt, [jnp.where(msk, ld, _DUMP)], ones16, mask=msk)
                return m + cs[15]
            m = lax.fori_loop(0, _C // 16, scan, 0)

            msrc[pl.ds(m, 16)] = zi
            mldst[pl.ds(m, 16)] = dv
            mpad = ((m + 7) // 8) * 8

            moff = wid * _LCAP + pl.multiple_of(mtot, 8)
            pltpu.async_copy(msrc.at[pl.ds(0, _C)],
                             ms_hbm.at[pl.ds(moff, _C)], fs0)
            pltpu.async_copy(mldst.at[pl.ds(0, _C)],
                             ml_hbm.at[pl.ds(moff, _C)], fs1)
            return mtot + mpad
        mtot = lax.fori_loop(0, _NCH, chunk_body, 0)
        pltpu.make_async_copy(msrc.at[pl.ds(0, _C)],
                              ms_hbm.at[pl.ds(0, _C)], fs0).wait()
        pltpu.make_async_copy(mldst.at[pl.ds(0, _C)],
                              ml_hbm.at[pl.ds(0, _C)], fs1).wait()

        cstage[pl.ds(0, 16)] = jnp.full((16,), 1, jnp.int32) * mtot
        pltpu.sync_copy(cstage, mc_hbm.at[pl.ds(wid * 16, 16)])
        pltpu.sync_copy(acnt.at[pl.ds(0, _NPT)], cnt_hbm.at[pl.ds(base, _NPT)])

    return body


def _layer_kernel(fp):
    acc_n = _ACC_ROWS * fp
    mesh = plsc.VectorSubcoreMesh(core_axis_name="c", subcore_axis_name="s")
    out_type = [jax.ShapeDtypeStruct((_NPAD * fp,), jnp.float32)] * 4
    scratch = [
        pltpu.VMEM((2, _G), jnp.int32),
        pltpu.VMEM((2, _G + 16), jnp.int32),
        pltpu.VMEM((2, _G, 128), jnp.float32),
        pltpu.VMEM((16,), jnp.int32),
        pltpu.VMEM((acc_n,), jnp.float32),
        pltpu.VMEM((acc_n,), jnp.float32),
        pltpu.VMEM((acc_n,), jnp.float32),
        pltpu.VMEM((acc_n,), jnp.float32),
        pltpu.SemaphoreType.DMA,
        pltpu.SemaphoreType.DMA,
        pltpu.SemaphoreType.DMA,
        pltpu.SemaphoreType.DMA,
        pltpu.SemaphoreType.DMA,
        pltpu.SemaphoreType.DMA,
    ]

    @functools.partial(pl.kernel, out_type=out_type, mesh=mesh,
                       scratch_types=scratch,
                       compiler_params=pltpu.CompilerParams(
                           needs_layout_passes=False))
    def body(b_hbm, ms_hbm, ml_hbm, mc_hbm, s1_hbm, s2_hbm, mx_hbm, mn_hbm,
             ib, lb, gbuf, cstage, a1, a2, amx, amn,
             is0, is1, ls0, ls1, gs0, gs1):
        wid = lax.axis_index("s") * _NC + lax.axis_index("c")
        base = wid * _NPT
        zf = jnp.zeros((16,), jnp.float32)
        neg = jnp.full((16,), _NEG, jnp.float32)
        pos16 = jnp.full((16,), _POS, jnp.float32)

        def init(i, _):
            a1[pl.ds(i * 16, 16)] = zf
            a2[pl.ds(i * 16, 16)] = zf
            amx[pl.ds(i * 16, 16)] = neg
            amn[pl.ds(i * 16, 16)] = pos16
            return 0
        lax.fori_loop(0, acc_n // 16, init, 0)

        pltpu.sync_copy(mc_hbm.at[pl.ds(wid * 16, 16)], cstage)
        mt = cstage[pl.ds(0, 16)][0]
        nb = (mt + _G - 1) // _G

        def i_start0(bi):
            pltpu.async_copy(ms_hbm.at[pl.ds(wid * _LCAP + bi * _G, _G)], ib.at[0], is0)
            pltpu.async_copy(ml_hbm.at[pl.ds(wid * _LCAP + bi * _G, _G)],
                             lb.at[0, pl.ds(0, _G)], ls0)

        def i_start1(bi):
            pltpu.async_copy(ms_hbm.at[pl.ds(wid * _LCAP + bi * _G, _G)], ib.at[1], is1)
            pltpu.async_copy(ml_hbm.at[pl.ds(wid * _LCAP + bi * _G, _G)],
                             lb.at[1, pl.ds(0, _G)], ls1)

        def i_wait0(bi):
            pltpu.make_async_copy(ms_hbm.at[pl.ds(wid * _LCAP + bi * _G, _G)],
                                  ib.at[0], is0).wait()
            pltpu.make_async_copy(ml_hbm.at[pl.ds(wid * _LCAP + bi * _G, _G)],
                                  lb.at[0, pl.ds(0, _G)], ls0).wait()

        def i_wait1(bi):
            pltpu.make_async_copy(ms_hbm.at[pl.ds(wid * _LCAP + bi * _G, _G)],
                                  ib.at[1], is1).wait()
            pltpu.make_async_copy(ml_hbm.at[pl.ds(wid * _LCAP + bi * _G, _G)],
                                  lb.at[1, pl.ds(0, _G)], ls1).wait()

        def g_start0():
            pltpu.async_copy(b_hbm.at[ib.at[0]], gbuf.at[0], gs0)

        def g_start1():
            pltpu.async_copy(b_hbm.at[ib.at[1]], gbuf.at[1], gs1)

        def g_wait0():
            pltpu.make_async_copy(b_hbm.at[ib.at[0]], gbuf.at[0], gs0).wait()

        def g_wait1():
            pltpu.make_async_copy(b_hbm.at[ib.at[1]], gbuf.at[1], gs1).wait()

        def rows(bi, s):
            nr = jnp.minimum(mt - bi * _G, _G)

            def row(r, ld):
                ld_next = lb[s, pl.ds(r + 1, 16)][0]
                off = ld * fp
                for c in range(fp // 16):
                    g = gbuf[s, r, pl.ds(c * 16, 16)]
                    o = off + c * 16
                    plsc.addupdate(a1.at[pl.ds(o, 16)], g)
                    plsc.addupdate(a2.at[pl.ds(o, 16)], g * g)
                    amx[pl.ds(o, 16)] = jnp.maximum(amx[pl.ds(o, 16)], g)
                    amn[pl.ds(o, 16)] = jnp.minimum(amn[pl.ds(o, 16)], g)
                return ld_next
            lax.fori_loop(0, nr, row, lb[s, pl.ds(0, 16)][0])

        @pl.when(nb > 0)
        def _():
            i_start0(0)
            i_wait0(0)
            g_start0()

            @pl.when(nb > 1)
            def _():
                i_start1(1)

        def pairs(j, _):
            b0 = 2 * j

            @pl.when(b0 + 1 < nb)
            def _():
                i_wait1(b0 + 1)
                g_start1()
            g_wait0()
            rows(b0, 0)

            @pl.when(b0 + 2 < nb)
            def _():
                i_start0(b0 + 2)

            @pl.when(b0 + 1 < nb)
            def _():
                g_wait1()
                rows(b0 + 1, 1)

            @pl.when(b0 + 3 < nb)
            def _():
                i_start1(b0 + 3)

            @pl.when(b0 + 2 < nb)
            def _():
                i_wait0(b0 + 2)
                g_start0()
            return 0
        lax.fori_loop(0, (nb + 1) // 2, pairs, 0)

        pltpu.sync_copy(a1.at[pl.ds(0, _NPT * fp)],
                        s1_hbm.at[pl.ds(base * fp, _NPT * fp)])
        pltpu.sync_copy(a2.at[pl.ds(0, _NPT * fp)],
                        s2_hbm.at[pl.ds(base * fp, _NPT * fp)])
        pltpu.sync_copy(amx.at[pl.ds(0, _NPT * fp)],
                        mx_hbm.at[pl.ds(base * fp, _NPT * fp)])
        pltpu.sync_copy(amn.at[pl.ds(0, _NPT * fp)],
                        mn_hbm.at[pl.ds(base * fp, _NPT * fp)])

    return body


_prep = _prep_kernel()
_layer = {80: _layer_kernel(80), 64: _layer_kernel(64)}


_RB = 1024
_NRB = _NPAD // _RB


def _proj_body(x_ref, wd, bd, ws, a_ref, b_ref):
    x = x_ref[...]
    a_ref[...] = jnp.dot(x, wd[...], preferred_element_type=jnp.float32) + bd[...]
    bm = jnp.dot(x, ws[...], preferred_element_type=jnp.float32)
    b_ref[...] = jnp.pad(bm, ((0, 0), (0, 128 - bm.shape[1])))


def _proj(xpad, p, f):
    return pl.pallas_call(
        _proj_body,
        grid=(_NRB,),
        in_specs=[pl.BlockSpec((_RB, f), lambda i: (i, 0)),
                  pl.BlockSpec((f, f), lambda i: (0, 0)),
                  pl.BlockSpec((1, f), lambda i: (0, 0)),
                  pl.BlockSpec((f, f), lambda i: (0, 0))],
        out_specs=[pl.BlockSpec((_RB, f), lambda i: (i, 0)),
                   pl.BlockSpec((_RB, 128), lambda i: (i, 0))],
        out_shape=[jax.ShapeDtypeStruct((_NPAD, f), jnp.float32),
                   jax.ShapeDtypeStruct((_NPAD, 128), jnp.float32)],
    )(xpad, p['pre_W'][:f], p['pre_b'][None, :], p['pre_W'][f:])


def _combine_body(x_ref, a_ref, s1_ref, s2_ref, mx_ref, mn_ref, cnt_ref,
                  deg_ref, wx, wall, pb, lw, lb_, bng, bnb,
                  wd, bd, ws, *out_refs, f, fn, last):
    h_ref = out_refs[0]
    x = x_ref[...]
    a = a_ref[...]
    s1 = s1_ref[...][:, :f]
    s2 = s2_ref[...][:, :f]
    mx = mx_ref[...][:, :f]
    mn = mn_ref[...][:, :f]
    c = cnt_ref[...]
    d = deg_ref[...]
    mean = (c * a + s1) / d
    mean_sq = (c * a * a + 2.0 * a * s1 + s2) / d
    std = jnp.sqrt(jnp.maximum(mean_sq - mean * mean, 0.0) + 1e-5)
    mxo = jnp.where(c > 0, a + mx, 0.0)
    mno = jnp.where(c > 0, a + mn, 0.0)
    agg = jnp.concatenate([mean, mxo, mno, std], axis=-1)
    logd = jnp.log(d + 1.0)
    p3 = jnp.dot(agg, wall[...], preferred_element_type=jnp.float32)
    out = (jnp.dot(x, wx[...], preferred_element_type=jnp.float32)
           + p3[:, :64] + (logd / AVG_LOG) * p3[:, 64:128]
           + (AVG_LOG / logd) * p3[:, 128:192] + pb[...])
    out = jnp.dot(out, lw[...], preferred_element_type=jnp.float32) + lb_[...]
    h = jax.nn.relu(out / np.sqrt(1.0 + 1e-5) * bng[...] + bnb[...])
    h_ref[...] = h
    if not last:
        an_ref, bn_ref = out_refs[1], out_refs[2]
        an_ref[...] = (jnp.dot(h, wd[...], preferred_element_type=jnp.float32)
                       + bd[...])
        bm = jnp.dot(h, ws[...], preferred_element_type=jnp.float32)
        bn_ref[...] = jnp.pad(bm, ((0, 0), (0, 128 - bm.shape[1])))


def _combine(xpad, a, aggs, cnt2, deg2, p, pnext, last):
    f = xpad.shape[1]
    fp = 80 if f == 78 else 64
    w = p['post_W']
    wall = jnp.concatenate([w[f:f + 4 * f], w[f + 4 * f:f + 8 * f],
                            w[f + 8 * f:]], axis=1)
    fn = 64
    outs = [jax.ShapeDtypeStruct((_NPAD, 64), jnp.float32)]
    out_specs = [pl.BlockSpec((_RB, 64), lambda i: (i, 0))]
    if not last:
        outs += [jax.ShapeDtypeStruct((_NPAD, fn), jnp.float32),
                 jax.ShapeDtypeStruct((_NPAD, 128), jnp.float32)]
        out_specs += [pl.BlockSpec((_RB, fn), lambda i: (i, 0)),
                      pl.BlockSpec((_RB, 128), lambda i: (i, 0))]
    wd = pnext['pre_W'][:fn] if not last else jnp.zeros((64, fn), jnp.float32)
    bd = (pnext['pre_b'][None, :] if not last
          else jnp.zeros((1, fn), jnp.float32))
    ws = pnext['pre_W'][fn:] if not last else jnp.zeros((64, fn), jnp.float32)
    return pl.pallas_call(
        functools.partial(_combine_body, f=f, fn=fn, last=last),
        grid=(_NRB,),
        in_specs=[pl.BlockSpec((_RB, f), lambda i: (i, 0)),
                  pl.BlockSpec((_RB, f), lambda i: (i, 0))]
        + [pl.BlockSpec((_RB, fp), lambda i: (i, 0))] * 4
        + [pl.BlockSpec((_RB, 1), lambda i: (i, 0))] * 2
        + [pl.BlockSpec((f, 64), lambda i: (0, 0)),
           pl.BlockSpec((4 * f, 192), lambda i: (0, 0)),
           pl.BlockSpec((1, 64), lambda i: (0, 0)),
           pl.BlockSpec((64, 64), lambda i: (0, 0)),
           pl.BlockSpec((1, 64), lambda i: (0, 0)),
           pl.BlockSpec((1, 64), lambda i: (0, 0)),
           pl.BlockSpec((1, 64), lambda i: (0, 0)),
           pl.BlockSpec((64, fn), lambda i: (0, 0)),
           pl.BlockSpec((1, fn), lambda i: (0, 0)),
           pl.BlockSpec((64, fn), lambda i: (0, 0))],
        out_specs=out_specs,
        out_shape=outs,
    )(xpad, a, aggs[0], aggs[1], aggs[2], aggs[3], cnt2, deg2,
      w[:f], wall, p['post_b'][None, :], p['lin_W'], p['lin_b'][None, :],
      p['bn_g'][None, :], p['bn_b'][None, :], wd, bd, ws)


_NB_POOL = _NPAD // 1024


def _pool_body(batch_ref, h_ref, pool_ref, gcnt_ref):
    i = pl.program_id(0)

    @pl.when(i == 0)
    def _():
        pool_ref[...] = jnp.zeros_like(pool_ref)
        gcnt_ref[...] = jnp.zeros_like(gcnt_ref)

    iot = jax.lax.broadcasted_iota(jnp.int32, (1024, B), 1)
    onehot = (batch_ref[...] == iot).astype(jnp.float32)
    pool_ref[...] += jnp.dot(onehot.T, h_ref[...],
                             preferred_element_type=jnp.float32)
    gcnt_ref[...] += jnp.sum(onehot, axis=0)[None, :]


def _pool(batch_b2, h_pad):
    return pl.pallas_call(
        _pool_body,
        grid=(_NB_POOL,),
        in_specs=[pl.BlockSpec((1024, B), lambda i: (i, 0)),
                  pl.BlockSpec((1024, 64), lambda i: (i, 0))],
        out_specs=[pl.BlockSpec((B, 64), lambda i: (0, 0)),
                   pl.BlockSpec((1, B), lambda i: (0, 0))],
        out_shape=[jax.ShapeDtypeStruct((B, 64), jnp.float32),
                   jax.ShapeDtypeStruct((1, B), jnp.float32)],
    )(batch_b2, h_pad)


def _mlp_body(pool_ref, gcnt_ref, xt_ref, wg, bg, w1a, w1b, b1, w2, b2,
              w3, b3, w4, b4, out_ref):
    gc = jnp.maximum(gcnt_ref[...].reshape(B, 1), 1.0)
    xg = pool_ref[...] / gc
    xg = jax.nn.relu(jnp.dot(xg, wg[...],
                             preferred_element_type=jnp.float32) + bg[...])
    h = jax.nn.relu(jnp.dot(xg, w1a[...], preferred_element_type=jnp.float32)
                    + jnp.dot(xt_ref[...], w1b[...],
                              preferred_element_type=jnp.float32) + b1[...])
    h = jax.nn.relu(jnp.dot(h, w2[...],
                            preferred_element_type=jnp.float32) + b2[...])
    h = jax.nn.relu(jnp.dot(h, w3[...],
                            preferred_element_type=jnp.float32) + b3[...])
    out_ref[...] = jnp.dot(h, w4[...],
                           preferred_element_type=jnp.float32) + b4[...]


def _mlp_head(pool, gcnt, xt, params):
    return pl.pallas_call(
        _mlp_body,
        out_shape=jax.ShapeDtypeStruct((B, 1), jnp.float32),
    )(pool, gcnt, xt,
      params['fc1_xd_W'], params['fc1_xd_b'][None, :],
      params['fc1_W'][:128], params['fc1_W'][128:], params['fc1_b'][None, :],
      params['fc2_W'], params['fc2_b'][None, :],
      params['fc3_W'], params['fc3_b'][None, :],
      params['out_W'], params['out_b'][None, :])


def _prot_body(tgt_ref, emb_ref, w1_ref, b1_ref, w2_ref, b2_ref,
               w3_ref, b3_ref, wp_ref, bp_ref, out_ref):
    emb = emb_ref[...]
    iot27 = jax.lax.broadcasted_iota(jnp.int32, (1000, 27), 1)
    for j in range(8):
        tok = tgt_ref[j]
        oh = (tok[:, None] == iot27).astype(jnp.float32)
        e = jnp.dot(oh, emb, preferred_element_type=jnp.float32)
        p = jnp.dot(e, w1_ref[...], preferred_element_type=jnp.float32)
        o1 = b1_ref[...]
        for k in range(8):
            o1 = o1 + p[k:k + 993, k * 32:(k + 1) * 32]
        o1 = jax.nn.relu(o1)
        x2 = jnp.concatenate([o1[k:k + 986, :] for k in range(8)], axis=1)
        o2 = jax.nn.relu(jnp.dot(x2, w2_ref[...],
                                 preferred_element_type=jnp.float32)
                         + b2_ref[...])
        x3 = jnp.concatenate([o2[k:k + 979, :] for k in range(8)], axis=1)
        o3 = jax.nn.relu(jnp.dot(x3, w3_ref[...],
                                 preferred_element_type=jnp.float32)
                         + b3_ref[...])
        mx = jnp.max(o3, axis=0, keepdims=True)
        out_ref[j, :] = (jnp.dot(mx, wp_ref[...],
                                 preferred_element_type=jnp.float32)
                         + bp_ref[...])[0]


def _protein(target, params):
    w1 = jnp.transpose(params['c1_W'], (1, 2, 0)).reshape(128, 256)
    b1 = jnp.broadcast_to(params['c1_b'][None, :], (993, 32))
    w2 = jnp.transpose(params['c2_W'], (2, 1, 0)).reshape(256, 64)
    b2 = params['c2_b'][None, :]
    w3 = jnp.transpose(params['c3_W'], (2, 1, 0)).reshape(512, 96)
    b3 = params['c3_b'][None, :]
    return pl.pallas_call(
        _prot_body,
        grid=(B // 8,),
        in_specs=[pl.BlockSpec((8, 1000), lambda i: (i, 0))]
        + [pl.BlockSpec(s, lambda i: tuple([0] * len(s)))
           for s in [(27, 128), (128, 256), (993, 32), (256, 64), (1, 64),
                     (512, 96), (1, 96), (96, 128), (1, 128)]],
        out_specs=pl.BlockSpec((8, 128), lambda i: (i, 0)),
        out_shape=jax.ShapeDtypeStruct((B, 128), jnp.float32),
    )(target, params['emb'], w1, b1, w2, b2, w3, b3,
      params['pfc_W'], params['pfc_b'][None, :])


def kernel(x, edge_index, batch, target, params):
    src, dst = edge_index[0], edge_index[1]
    msl, mll, mcl, cntf = _prep(src, dst)
    cnt2 = cntf[:, None]
    deg2 = jnp.maximum(cntf, 1.0)[:, None]

    xpad = jnp.zeros((_NPAD, 78), jnp.float32).at[:N_NODES].set(x)
    a1, b1 = _proj(xpad, params['conv1'], 78)
    aggs1 = [o.reshape(_NPAD, 80) for o in _layer[80](b1, msl, mll, mcl)]
    h1, a2, b2 = _combine(xpad, a1, aggs1, cnt2, deg2,
                          params['conv1'], params['conv2'], False)
    aggs2 = [o.reshape(_NPAD, 64) for o in _layer[64](b2, msl, mll, mcl)]
    h2, a3, b3 = _combine(h1, a2, aggs2, cnt2, deg2,
                          params['conv2'], params['conv3'], False)
    aggs3 = [o.reshape(_NPAD, 64) for o in _layer[64](b3, msl, mll, mcl)]
    (h3,) = _combine(h2, a3, aggs3, cnt2, deg2,
                     params['conv3'], None, True)

    batch_pad = jnp.full((_NPAD,), 999, jnp.int32).at[:N_NODES].set(batch)
    batch_b2 = jnp.broadcast_to(batch_pad[:, None], (_NPAD, B))
    pool, gcnt = _pool(batch_b2, h3)

    xt = _protein(target, params)
    return _mlp_head(pool, gcnt, xt, params)
